# baseline (speedup 1.0000x reference)
import functools

import jax
import jax.numpy as jnp
from jax import lax
from jax.experimental import pallas as pl
from jax.experimental.pallas import tpu as pltpu
from jax.experimental.pallas import tpu_sc as plsc

_D = 32
_BT = 128
_NBUF = 4


@functools.cache
def _make_gather(hist, batch, vocab):
    info = plsc.get_sparse_core_info()
    nc, ns = info.num_cores, info.num_subcores
    nw = nc * ns
    bslice = batch // nw
    ntile = bslice // _BT
    nunit = hist * ntile
    ngroup = nunit // _NBUF
    assert bslice * nw == batch and ntile * _BT == bslice
    assert ngroup * _NBUF == nunit

    mesh = plsc.VectorSubcoreMesh(core_axis_name="c", subcore_axis_name="s")
    out_sds = jax.ShapeDtypeStruct(
        (hist, _D // 8, batch // _BT, 8 * _BT), jnp.float32)
    rows_t = pltpu.VMEM((_BT, _D), jnp.float32)
    tile_t = pltpu.VMEM((_D * _BT,), jnp.float32)

    @functools.partial(
        pl.kernel,
        mesh=mesh,
        compiler_params=pltpu.CompilerParams(
            use_tc_tiling_on_sc=False, needs_layout_passes=False),
        out_type=(out_sds, out_sds),
        scratch_types=(
            [pltpu.VMEM((2, hist, bslice), jnp.int32)]
            + [rows_t] * _NBUF
            + [tile_t] * _NBUF
            + [pltpu.SemaphoreType.DMA((_NBUF,)),
               pltpu.SemaphoreType.DMA((_NBUF,))]
        ),
    )
    def gather2(table_hbm, idx_a_hbm, idx_b_hbm, out_a_hbm, out_b_hbm,
                idx_v, *refs):
        rows = refs[:_NBUF]
        tbuf = refs[_NBUF:2 * _NBUF]
        gsem, ssem = refs[2 * _NBUF], refs[2 * _NBUF + 1]
        wid = lax.axis_index("s") * nc + lax.axis_index("c")
        b0 = wid * bslice
        pltpu.sync_copy(idx_a_hbm.at[:, pl.ds(b0, bslice)], idx_v.at[0])
        pltpu.sync_copy(idx_b_hbm.at[:, pl.ds(b0, bslice)], idx_v.at[1])

        def run_domain(dom, out_hbm):
            def gather_op(u, b):
                h = u // ntile
                t = u - h * ntile
                return pltpu.make_async_copy(
                    table_hbm.at[idx_v.at[dom, h, pl.ds(t * _BT, _BT)]],
                    rows[b], gsem.at[b])

            def store_ops(u, b):
                h = u // ntile
                t = u - h * ntile
                return [
                    pltpu.make_async_copy(
                        tbuf[b].at[pl.ds(ft * 8 * _BT, 8 * _BT)],
                        out_hbm.at[h, ft, wid * ntile + t], ssem.at[b])
                    for ft in range(_D // 8)
                ]

            def transpose(b):
                rows_b, tbuf_b = rows[b], tbuf[b]

                def per_j4(j4, carry):
                    for jj in range(4):
                        j = j4 * 4 + jj
                        lane = lax.iota(jnp.int32, 16)
                        jv = jnp.broadcast_to(j, (16,)).astype(jnp.int32)
                        for k in range(_D // 16):
                            vals = rows_b[j, pl.ds(k * 16, 16)]
                            plsc.store_scatter(
                                tbuf_b, [(k * 16 + lane) * _BT + jv], vals)
                    return carry

                lax.fori_loop(0, _BT // 4, per_j4, 0)

            for b in range(_NBUF):
                gather_op(b, b).start()

            def group(g, carry):
                for b in range(_NBUF):
                    u = g * _NBUF + b
                    gather_op(u, b).wait()

                    @pl.when(g > 0)
                    def _():
                        for op in store_ops(u - _NBUF, b):
                            op.wait()

                    transpose(b)
                    for op in store_ops(u, b):
                        op.start()

                    @pl.when(g < ngroup - 1)
                    def _():
                        gather_op(u + _NBUF, b).start()

                return carry

            lax.fori_loop(0, ngroup, group, 0)
            for b in range(_NBUF):
                for op in store_ops(nunit - _NBUF + b, b):
                    op.wait()

        run_domain(0, out_a_hbm)
        run_domain(1, out_b_hbm)

    return gather2


def kernel(indices_domain_a, indices_domain_b, table):
    batch, hist = indices_domain_a.shape
    idx_at = indices_domain_a.T.astype(jnp.int32)
    idx_bt = indices_domain_b.T.astype(jnp.int32)
    out_a, out_b = _make_gather(hist, batch, table.shape[0])(
        table, idx_at, idx_bt)

    def form(o4):
        o5 = o4.reshape(hist, _D // 8, batch // _BT, 8, _BT)
        return jnp.transpose(o5, (2, 4, 0, 1, 3)).reshape(batch, hist, _D)

    return (indices_domain_a, form(out_a), indices_domain_b, form(out_b))


# speedup vs baseline: 1.1647x; 1.1647x over previous
import functools

import jax
import jax.numpy as jnp
from jax import lax
from jax.experimental import pallas as pl
from jax.experimental.pallas import tpu as pltpu
from jax.experimental.pallas import tpu_sc as plsc

_D = 32
_BT = 128
_NBUF = 4


@functools.cache
def _make_gather(hist, batch, vocab):
    info = plsc.get_sparse_core_info()
    nc, ns = info.num_cores, info.num_subcores
    nw = nc * ns
    bslice = batch // nw
    ntile = bslice // _BT
    nunit = hist * ntile
    ngroup = nunit // _NBUF
    assert bslice * nw == batch and ntile * _BT == bslice
    assert ngroup * _NBUF == nunit

    mesh = plsc.VectorSubcoreMesh(core_axis_name="c", subcore_axis_name="s")
    out_sds = jax.ShapeDtypeStruct(
        (hist, _D // 8, batch // _BT, 8 * _BT), jnp.float32)
    rows_t = pltpu.VMEM((_BT, _D), jnp.float32)
    tile_t = pltpu.VMEM((_D * _BT,), jnp.float32)
    _PS = _D + 1

    @functools.partial(
        pl.kernel,
        mesh=mesh,
        compiler_params=pltpu.CompilerParams(
            use_tc_tiling_on_sc=False, needs_layout_passes=False),
        out_type=(out_sds, out_sds),
        scratch_types=(
            [pltpu.VMEM((2, hist, bslice), jnp.int32)]
            + [rows_t] * _NBUF
            + [tile_t] * _NBUF
            + [pltpu.VMEM((_BT * _PS,), jnp.float32)]
            + [pltpu.SemaphoreType.DMA((_NBUF,)),
               pltpu.SemaphoreType.DMA((_NBUF,))]
        ),
    )
    def gather2(table_hbm, idx_a_hbm, idx_b_hbm, out_a_hbm, out_b_hbm,
                idx_v, *refs):
        rows = refs[:_NBUF]
        tbuf = refs[_NBUF:2 * _NBUF]
        pad_v = refs[2 * _NBUF]
        gsem, ssem = refs[2 * _NBUF + 1], refs[2 * _NBUF + 2]
        wid = lax.axis_index("s") * nc + lax.axis_index("c")
        b0 = wid * bslice
        pltpu.sync_copy(idx_a_hbm.at[:, pl.ds(b0, bslice)], idx_v.at[0])
        pltpu.sync_copy(idx_b_hbm.at[:, pl.ds(b0, bslice)], idx_v.at[1])

        def run_domain(dom, out_hbm):
            def gather_op(u, b):
                h = u // ntile
                t = u - h * ntile
                return pltpu.make_async_copy(
                    table_hbm.at[idx_v.at[dom, h, pl.ds(t * _BT, _BT)]],
                    rows[b], gsem.at[b])

            def store_ops(u, b):
                h = u // ntile
                t = u - h * ntile
                return [
                    pltpu.make_async_copy(
                        tbuf[b].at[pl.ds(ft * 8 * _BT, 8 * _BT)],
                        out_hbm.at[h, ft, wid * ntile + t], ssem.at[b])
                    for ft in range(_D // 8)
                ]

            def transpose(b):
                rows_b, tbuf_b = rows[b], tbuf[b]

                def per_j4(j4, carry):
                    for jj in range(4):
                        j = j4 * 4 + jj
                        lane = lax.iota(jnp.int32, 16)
                        base = jnp.broadcast_to(j * _PS, (16,)).astype(
                            jnp.int32)
                        for k in range(_D // 16):
                            vals = rows_b[j, pl.ds(k * 16, 16)]
                            plsc.store_scatter(
                                pad_v, [base + (k * 16 + lane)], vals)
                    return carry

                lax.fori_loop(0, _BT // 4, per_j4, 0)

                def per_f(f, carry):
                    for jb in range(_BT // 16):
                        lane = lax.iota(jnp.int32, 16)
                        addr = (jb * 16 + lane) * _PS + f
                        vals = plsc.load_gather(pad_v, [addr])
                        tbuf_b[pl.ds(f * _BT + jb * 16, 16)] = vals
                    return carry

                lax.fori_loop(0, _D, per_f, 0)

            for b in range(_NBUF):
                gather_op(b, b).start()

            def group(g, carry):
                for b in range(_NBUF):
                    u = g * _NBUF + b
                    gather_op(u, b).wait()

                    @pl.when(g > 0)
                    def _():
                        for op in store_ops(u - _NBUF, b):
                            op.wait()

                    transpose(b)
                    for op in store_ops(u, b):
                        op.start()

                    @pl.when(g < ngroup - 1)
                    def _():
                        gather_op(u + _NBUF, b).start()

                return carry

            lax.fori_loop(0, ngroup, group, 0)
            for b in range(_NBUF):
                for op in store_ops(nunit - _NBUF + b, b):
                    op.wait()

        run_domain(0, out_a_hbm)
        run_domain(1, out_b_hbm)

    return gather2


def kernel(indices_domain_a, indices_domain_b, table):
    batch, hist = indices_domain_a.shape
    idx_at = indices_domain_a.T.astype(jnp.int32)
    idx_bt = indices_domain_b.T.astype(jnp.int32)
    out_a, out_b = _make_gather(hist, batch, table.shape[0])(
        table, idx_at, idx_bt)

    def form(o4):
        o5 = o4.reshape(hist, _D // 8, batch // _BT, 8, _BT)
        return jnp.transpose(o5, (2, 4, 0, 1, 3)).reshape(batch, hist, _D)

    return (indices_domain_a, form(out_a), indices_domain_b, form(out_b))
